# baseline (speedup 1.0000x reference)
import functools

import jax
import jax.numpy as jnp
import numpy as np
from jax import lax
from jax.experimental import pallas as pl
from jax.experimental.pallas import tpu as pltpu
from jax.experimental.pallas import tpu_sc as plsc

L = 16
NW = 16
N = 20000
NPAD = 20480
NA = NPAD // NW
NV = NA // L
G = 50
GP = 64
NR = 10
HB = 256

POS_T = 0.7
NEG_T = 0.3
TOTAL = 256
MAX_POS = 128
SIG2 = 9.0

_LN_COEFS = (
    -0.0022883228657252968, 0.038030295273843794, -0.2864361250512785,
    1.2917075421662867, -3.8809206183156606, 8.178308102497969,
    -12.396895192830529, 13.666792234339184, -11.06002906824556,
    7.031391849388096, -2.5796606939698807,
)
_LN2 = 0.6931471805599453


def _ln_12(x):
    acc = jnp.full((L,), _LN_COEFS[0], jnp.float32)
    for c in _LN_COEFS[1:]:
        acc = acc * x + jnp.float32(c)
    return acc


def _ln_pos(x):
    bits = plsc.bitcast(x, jnp.int32)
    e = ((bits >> 23) & 0xFF) - 127
    mant = plsc.bitcast((bits & 0x7FFFFF) | 0x3F800000, jnp.float32)
    return e.astype(jnp.float32) * jnp.float32(_LN2) + _ln_12(mant)


def _iota():
    return lax.broadcasted_iota(jnp.int32, (L,), 0)


def _walk(histsum, t, ascending):
    def body(it, carry):
        cnt, bacc, sacc = carry
        c = it if ascending else (HB // L - 1) - it
        chunk = histsum[pl.ds(c * L, L)]
        if ascending:
            within = jnp.cumsum(chunk) - chunk
        else:
            within = lax.rev(jnp.cumsum(lax.rev(chunk, (0,))), (0,)) - chunk
        s = within + cnt
        is_b = (s < t) & (s + chunk >= t)
        bacc = bacc + jnp.where(is_b, c * L + _iota(), 0)
        sacc = sacc + jnp.where(is_b, s, 0)
        cnt = cnt + jnp.sum(chunk)
        return cnt, bacc, sacc

    zero = jnp.zeros((L,), jnp.int32)
    _, bacc, sacc = lax.fori_loop(0, HB // L, body,
                                  (jnp.int32(0), zero, zero))
    return jnp.sum(bacc), jnp.sum(sacc)


def _radix_round(r, wid, get_cand, sh, top, ascending, pref, t,
                 hist2d, histsum, rdhist, sthist):
    zero = jnp.zeros((L,), jnp.int32)
    ones = jnp.ones((L,), jnp.int32)

    def zbody(i, _):
        hist2d[pl.ds(i * L, L)] = zero
        return 0
    lax.fori_loop(0, L * HB // L, zbody, 0)

    io = _iota()

    def scan(v, _):
        mask, key = get_cand(v)
        if not top:
            mask = mask & ((key >> (sh + 8)) == (pref >> (sh + 8)))
        bucket = (key >> sh) & (HB - 1)
        slot = io * HB + bucket
        plsc.addupdate_scatter(hist2d, [slot], ones, mask=mask)
        return 0
    lax.fori_loop(0, NV, scan, 0)

    def lred_c(c, _):
        def lred_l(l, acc):
            return acc + hist2d[pl.ds(l * HB + c * L, L)]
        acc = lax.fori_loop(0, L, lred_l, zero)
        histsum[pl.ds(c * L, L)] = acc
        return 0
    lax.fori_loop(0, HB // L, lred_c, 0)

    pltpu.sync_copy(histsum, sthist.at[pl.ds(r * NW * HB + wid * HB, HB)])
    plsc.subcore_barrier()
    pltpu.sync_copy(sthist.at[pl.ds(r * NW * HB, NW * HB)], rdhist)

    def gred_c(c, _):
        def gred_i(i, acc):
            return acc + rdhist[pl.ds(i * HB + c * L, L)]
        histsum[pl.ds(c * L, L)] = lax.fori_loop(0, NW, gred_i, zero)
        return 0
    lax.fori_loop(0, HB // L, gred_c, 0)

    bstar, taken = _walk(histsum, t, ascending)
    return pref | (bstar << sh), t - taken


def _sc_kernel_body(ay0, ax0, ay1, ax1, s0h, s1h, p0h, p1h, p2h, p3h, rkh,
                    gy0s, gx0s, gy1s, gx1s, gabs, gy0c, gx0c, gy1c, gx1c, hwh,
                    out_hbm, st_gtmax, st_cnt, st_hist, st_loss,
                    cy0, cx0, cy1, cx1, vs0, vs1, vp0, vp1, vp2, vp3, vrk,
                    vgy0, vgx0, vgy1, vgx1, vgab, vg0c, vg1c, vg2c, vg3c, vhw,
                    iouf, gtmax, maxiou, bestj, insd, posm, negm, poskey,
                    hist2d, histsum, rdgt, rdhist, rdsm, gbs, wv, outv):
    wid = lax.axis_index("s")
    base = wid * NA
    fzero = jnp.zeros((L,), jnp.float32)
    io = _iota()

    pltpu.sync_copy(ay0.at[pl.ds(base, NA)], cy0)
    pltpu.sync_copy(ax0.at[pl.ds(base, NA)], cx0)
    pltpu.sync_copy(ay1.at[pl.ds(base, NA)], cy1)
    pltpu.sync_copy(ax1.at[pl.ds(base, NA)], cx1)
    pltpu.sync_copy(s0h.at[pl.ds(base, NA)], vs0)
    pltpu.sync_copy(s1h.at[pl.ds(base, NA)], vs1)
    pltpu.sync_copy(p0h.at[pl.ds(base, NA)], vp0)
    pltpu.sync_copy(p1h.at[pl.ds(base, NA)], vp1)
    pltpu.sync_copy(p2h.at[pl.ds(base, NA)], vp2)
    pltpu.sync_copy(p3h.at[pl.ds(base, NA)], vp3)
    pltpu.sync_copy(rkh.at[pl.ds(base, NA)], vrk)
    pltpu.sync_copy(gy0s, vgy0)
    pltpu.sync_copy(gx0s, vgx0)
    pltpu.sync_copy(gy1s, vgy1)
    pltpu.sync_copy(gx1s, vgx1)
    pltpu.sync_copy(gabs, vgab)
    pltpu.sync_copy(gy0c, vg0c)
    pltpu.sync_copy(gx0c, vg1c)
    pltpu.sync_copy(gy1c, vg2c)
    pltpu.sync_copy(gx1c, vg3c)
    pltpu.sync_copy(hwh, vhw)

    for j in range(G):
        gtmax[pl.ds(j * L, L)] = fzero - 1e30

    hvec = vhw[pl.ds(0, L)]
    wvec = vhw[pl.ds(L, L)]

    def p1_body(v, _):
        o = v * L
        a0 = cy0[pl.ds(o, L)]
        a1 = cx0[pl.ds(o, L)]
        a2 = cy1[pl.ds(o, L)]
        a3 = cx1[pl.ds(o, L)]
        ins = (a0 >= 0.0) & (a1 >= 0.0) & (a2 <= hvec) & (a3 <= wvec)
        area_a = (a2 - a0) * (a3 - a1)
        best = fzero - 1e30
        bj = jnp.zeros((L,), jnp.int32)
        for j in range(G):
            g0 = vgy0[pl.ds(j * L, L)]
            g1 = vgx0[pl.ds(j * L, L)]
            g2 = vgy1[pl.ds(j * L, L)]
            g3 = vgx1[pl.ds(j * L, L)]
            ab = vgab[pl.ds(j * L, L)]
            ih = jnp.maximum(jnp.minimum(a2, g2) - jnp.maximum(a0, g0), 0.0)
            iw = jnp.maximum(jnp.minimum(a3, g3) - jnp.maximum(a1, g1), 0.0)
            inter = ih * iw
            iou = inter / ((area_a + ab) - inter)
            iou = jnp.where(ins, iou, -1.0)
            gm = gtmax[pl.ds(j * L, L)]
            gtmax[pl.ds(j * L, L)] = jnp.maximum(gm, iou)
            upd = iou > best
            best = jnp.where(upd, iou, best)
            bj = jnp.where(upd, j, bj)
            iouf[pl.ds(j * NA + o, L)] = iou
        maxiou[pl.ds(o, L)] = best
        bestj[pl.ds(o, L)] = bj
        insd[pl.ds(o, L)] = jnp.where(ins, 1.0, 0.0)
        return 0

    lax.fori_loop(0, NV, p1_body, 0)

    pltpu.sync_copy(gtmax, st_gtmax.at[pl.ds(wid * G * L, G * L)])
    plsc.subcore_barrier()
    pltpu.sync_copy(st_gtmax, rdgt)

    for j in range(G):
        gbs[pl.ds(j * L, L)] = fzero - 1e30

    def gmerge(i, _):
        for j in range(G):
            cur = gbs[pl.ds(j * L, L)]
            gbs[pl.ds(j * L, L)] = jnp.maximum(
                cur, rdgt[pl.ds(i * G * L + j * L, L)])
        return 0
    lax.fori_loop(0, NW, gmerge, 0)

    for j in range(G):
        g = jnp.max(gbs[pl.ds(j * L, L)])
        gbs[pl.ds(j * L, L)] = fzero + g

    def p2_body(v, carry):
        cp_acc, cn_acc = carry
        o = v * L
        forced = io < 0
        for j in range(G):
            iv = iouf[pl.ds(j * NA + o, L)]
            gb = gbs[pl.ds(j * L, L)]
            forced = forced | ((iv >= gb) & (iv > 0.0))
        best = maxiou[pl.ds(o, L)]
        ins = insd[pl.ds(o, L)] > 0.5
        pm = ins & ((best >= POS_T) | forced)
        nm = ins & (best < NEG_T) & (best >= 0.0)
        posm[pl.ds(o, L)] = jnp.where(pm, 1.0, 0.0)
        negm[pl.ds(o, L)] = jnp.where(nm, 1.0, 0.0)
        key = plsc.bitcast(best, jnp.int32)
        poskey[pl.ds(o, L)] = jnp.where(pm, key, 0)
        return (cp_acc + jnp.where(pm, 1.0, 0.0),
                cn_acc + jnp.where(nm, 1.0, 0.0))

    cp_acc, cn_acc = lax.fori_loop(0, NV, p2_body, (fzero, fzero))
    wv[pl.ds(0, L)] = cp_acc
    wv[pl.ds(L, L)] = cn_acc
    pltpu.sync_copy(wv, st_cnt.at[pl.ds(wid * 2 * L, 2 * L)])
    plsc.subcore_barrier()
    pltpu.sync_copy(st_cnt, rdsm)

    def cmerge(i, carry):
        a, b = carry
        return (a + rdsm[pl.ds(i * 2 * L, L)],
                b + rdsm[pl.ds(i * 2 * L + L, L)])
    cpv, cnv = lax.fori_loop(0, NW, cmerge, (fzero, fzero))
    cnt_pos = jnp.sum(cpv)
    cnt_neg = jnp.sum(cnv)
    n_pos = jnp.minimum(cnt_pos, float(MAX_POS))
    need_f = float(TOTAL) - n_pos
    kfill_f = jnp.maximum(need_f - cnt_neg, 0.0)
    cp_i = cnt_pos.astype(jnp.int32)
    cn_i = cnt_neg.astype(jnp.int32)
    need_i = jnp.int32(TOTAL) - jnp.minimum(cp_i, MAX_POS)
    kfill_i = jnp.maximum(need_i - cn_i, 0)

    def get_pos(v):
        o = v * L
        return posm[pl.ds(o, L)] > 0.5, poskey[pl.ds(o, L)]

    def get_neg(v):
        o = v * L
        return negm[pl.ds(o, L)] > 0.5, vrk[pl.ds(o, L)]

    pref = jnp.int32(0)
    t = jnp.int32(MAX_POS)
    pref, t = _radix_round(0, wid, get_pos, 24, True, False, pref, t,
                           hist2d, histsum, rdhist, st_hist)
    pref, t = _radix_round(1, wid, get_pos, 16, False, False, pref, t,
                           hist2d, histsum, rdhist, st_hist)
    pref, t = _radix_round(2, wid, get_pos, 8, False, False, pref, t,
                           hist2d, histsum, rdhist, st_hist)
    pref, t = _radix_round(3, wid, get_pos, 0, False, False, pref, t,
                           hist2d, histsum, rdhist, st_hist)
    kv = pref
    t_rem = t

    def get_tie(v):
        o = v * L
        m = (posm[pl.ds(o, L)] > 0.5) & (poskey[pl.ds(o, L)] == kv)
        return m, base + o + io

    pref2 = jnp.int32(0)
    t2 = t_rem
    pref2, t2 = _radix_round(4, wid, get_tie, 8, True, True, pref2, t2,
                             hist2d, histsum, rdhist, st_hist)
    pref2, t2 = _radix_round(5, wid, get_tie, 0, False, True, pref2, t2,
                             hist2d, histsum, rdhist, st_hist)
    ki = pref2

    pref3 = jnp.int32(0)
    t3 = need_i
    pref3, t3 = _radix_round(6, wid, get_neg, 8, True, True, pref3, t3,
                             hist2d, histsum, rdhist, st_hist)
    pref3, t3 = _radix_round(7, wid, get_neg, 0, False, True, pref3, t3,
                             hist2d, histsum, rdhist, st_hist)
    kr = pref3

    def get_fill(v):
        o = v * L
        idx = base + o + io
        m = (negm[pl.ds(o, L)] <= 0.5) & (idx < N)
        return m, idx

    pref4 = jnp.int32(0)
    t4 = kfill_i
    pref4, t4 = _radix_round(8, wid, get_fill, 8, True, True, pref4, t4,
                             hist2d, histsum, rdhist, st_hist)
    pref4, t4 = _radix_round(9, wid, get_fill, 0, False, True, pref4, t4,
                             hist2d, histsum, rdhist, st_hist)
    kf = pref4

    pos_over = cnt_pos > float(MAX_POS)
    neg_over = cnt_neg > need_f

    def p4_body(v, carry):
        acc_cls, acc_reg = carry
        o = v * L
        idx = base + o + io
        pm = posm[pl.ds(o, L)] > 0.5
        nm = negm[pl.ds(o, L)] > 0.5
        key = poskey[pl.ds(o, L)]
        rk = vrk[pl.ds(o, L)]
        sel_pos_rad = (pm & (key > kv)) | (pm & (key == kv) & (idx <= ki))
        sel_pos = jnp.where(pos_over, sel_pos_rad, pm)
        sel_neg = jnp.where(neg_over, nm & (rk <= kr), nm)
        sel_fill = (kfill_i > 0) & (~nm) & (idx < N) & (idx <= kf)
        neg_w = sel_neg | sel_fill

        a0 = cy0[pl.ds(o, L)]
        a1 = cx0[pl.ds(o, L)]
        a2 = cy1[pl.ds(o, L)]
        a3 = cx1[pl.ds(o, L)]
        sv0 = vs0[pl.ds(o, L)]
        sv1 = vs1[pl.ds(o, L)]
        m = jnp.maximum(sv0, sv1)
        esum = jnp.exp(sv0 - m) + jnp.exp(sv1 - m)
        lse = m + _ln_12(esum)
        lp0 = sv0 - lse
        lp1 = sv1 - lse
        acc_cls = (acc_cls - jnp.where(sel_pos, lp1, 0.0)
                   - jnp.where(neg_w, lp0, 0.0))

        bj = bestj[pl.ds(o, L)]
        g0 = plsc.load_gather(vg0c, [bj])
        g1 = plsc.load_gather(vg1c, [bj])
        g2 = plsc.load_gather(vg2c, [bj])
        g3 = plsc.load_gather(vg3c, [bj])
        a_h = a2 - a0
        a_w = a3 - a1
        a_cy = a0 + 0.5 * a_h
        a_cx = a1 + 0.5 * a_w
        g_h = g2 - g0
        g_w = g3 - g1
        g_cy = g0 + 0.5 * g_h
        g_cx = g1 + 0.5 * g_w
        eps = jnp.float32(1e-8)
        ty = (g_cy - a_cy) / (a_h + eps)
        tx = (g_cx - a_cx) / (a_w + eps)
        th = _ln_pos(jnp.maximum(g_h, eps)) - _ln_pos(jnp.maximum(a_h, eps))
        tw = _ln_pos(jnp.maximum(g_w, eps)) - _ln_pos(jnp.maximum(a_w, eps))
        ssum = fzero
        for pref_ref, tgt in ((vp0, tx), (vp1, ty), (vp2, tw), (vp3, th)):
            d = pref_ref[pl.ds(o, L)] - tgt
            ad = jnp.abs(d)
            sl = jnp.where(ad < 1.0 / SIG2, 0.5 * SIG2 * d * d,
                           ad - 0.5 / SIG2)
            ssum = ssum + sl
        acc_reg = acc_reg + jnp.where(sel_pos, ssum, 0.0)
        return acc_cls, acc_reg

    acc_cls, acc_reg = lax.fori_loop(0, NV, p4_body, (fzero, fzero))
    wv[pl.ds(0, L)] = acc_cls
    wv[pl.ds(L, L)] = acc_reg
    pltpu.sync_copy(wv, st_loss.at[pl.ds(wid * 2 * L, 2 * L)])
    plsc.subcore_barrier()
    pltpu.sync_copy(st_loss, rdsm)

    def lmerge(i, carry):
        a, b = carry
        return (a + rdsm[pl.ds(i * 2 * L, L)],
                b + rdsm[pl.ds(i * 2 * L + L, L)])
    av, bv = lax.fori_loop(0, NW, lmerge, (fzero, fzero))
    cls_sum = jnp.sum(av)
    reg_sum = jnp.sum(bv)
    numer = (jnp.where(io == 0, cls_sum, 0.0)
             + jnp.where(io == 1, reg_sum, 0.0))
    denom = jnp.where(io == 1, jnp.maximum(n_pos, 1.0),
                      jnp.float32(TOTAL))
    outv[pl.ds(0, L)] = numer / denom

    @pl.when(wid == 0)
    def _():
        pltpu.sync_copy(outv, out_hbm)


def _build_call():
    mesh = plsc.VectorSubcoreMesh(core_axis_name="c", subcore_axis_name="s",
                                  num_cores=1, num_subcores=NW)
    f32, i32 = jnp.float32, jnp.int32
    return pl.kernel(
        _sc_kernel_body,
        out_type=[
            jax.ShapeDtypeStruct((L,), f32),
            jax.ShapeDtypeStruct((NW * G * L,), f32),
            jax.ShapeDtypeStruct((NW * 2 * L,), f32),
            jax.ShapeDtypeStruct((NR * NW * HB,), i32),
            jax.ShapeDtypeStruct((NW * 2 * L,), f32),
        ],
        mesh=mesh,
        compiler_params=pltpu.CompilerParams(needs_layout_passes=False),
        scratch_types=[
            pltpu.VMEM((NA,), f32),
            pltpu.VMEM((NA,), f32),
            pltpu.VMEM((NA,), f32),
            pltpu.VMEM((NA,), f32),
            pltpu.VMEM((NA,), f32),
            pltpu.VMEM((NA,), f32),
            pltpu.VMEM((NA,), f32),
            pltpu.VMEM((NA,), f32),
            pltpu.VMEM((NA,), f32),
            pltpu.VMEM((NA,), f32),
            pltpu.VMEM((NA,), i32),
            pltpu.VMEM((G * L,), f32),
            pltpu.VMEM((G * L,), f32),
            pltpu.VMEM((G * L,), f32),
            pltpu.VMEM((G * L,), f32),
            pltpu.VMEM((G * L,), f32),
            pltpu.VMEM((GP,), f32),
            pltpu.VMEM((GP,), f32),
            pltpu.VMEM((GP,), f32),
            pltpu.VMEM((GP,), f32),
            pltpu.VMEM((2 * L,), f32),
            pltpu.VMEM((G * NA,), f32),
            pltpu.VMEM((G * L,), f32),
            pltpu.VMEM((NA,), f32),
            pltpu.VMEM((NA,), i32),
            pltpu.VMEM((NA,), f32),
            pltpu.VMEM((NA,), f32),
            pltpu.VMEM((NA,), f32),
            pltpu.VMEM((NA,), i32),
            pltpu.VMEM((L * HB,), i32),
            pltpu.VMEM((HB,), i32),
            pltpu.VMEM((NW * G * L,), f32),
            pltpu.VMEM((NW * HB,), i32),
            pltpu.VMEM((NW * 2 * L,), f32),
            pltpu.VMEM((G * L,), f32),
            pltpu.VMEM((2 * L,), f32),
            pltpu.VMEM((L,), f32),
        ],
    )


def _pad1(x, value):
    return jnp.concatenate(
        [x, jnp.full((NPAD - N,), value, x.dtype)])


def kernel(image_shape, anchors, rpn_score, rpn_bboxes_txtytwth, gt_bboxes):
    f32 = jnp.float32
    rngv = jax.random.uniform(jax.random.key(123), (N,))
    order = jnp.argsort(-rngv, stable=True)
    rank = jnp.argsort(order, stable=True).astype(jnp.int32)

    ay0 = _pad1(anchors[:, 0].astype(f32), -1.0)
    ax0 = _pad1(anchors[:, 1].astype(f32), -1.0)
    ay1 = _pad1(anchors[:, 2].astype(f32), -1.0)
    ax1 = _pad1(anchors[:, 3].astype(f32), -1.0)
    s0 = _pad1(rpn_score[:, 0].astype(f32), 0.0)
    s1 = _pad1(rpn_score[:, 1].astype(f32), 0.0)
    p0 = _pad1(rpn_bboxes_txtytwth[:, 0].astype(f32), 0.0)
    p1 = _pad1(rpn_bboxes_txtytwth[:, 1].astype(f32), 0.0)
    p2 = _pad1(rpn_bboxes_txtytwth[:, 2].astype(f32), 0.0)
    p3 = _pad1(rpn_bboxes_txtytwth[:, 3].astype(f32), 0.0)
    rk = _pad1(rank, 32000)

    gt = gt_bboxes.astype(f32)
    gy0s = jnp.broadcast_to(gt[:, 0:1], (G, L)).reshape(-1)
    gx0s = jnp.broadcast_to(gt[:, 1:2], (G, L)).reshape(-1)
    gy1s = jnp.broadcast_to(gt[:, 2:3], (G, L)).reshape(-1)
    gx1s = jnp.broadcast_to(gt[:, 3:4], (G, L)).reshape(-1)
    ab = ((gt[:, 2] - gt[:, 0]) * (gt[:, 3] - gt[:, 1]) + 1e-9)
    gabs = jnp.broadcast_to(ab[:, None], (G, L)).reshape(-1)
    gpad = jnp.zeros((GP - G,), f32)
    gy0c = jnp.concatenate([gt[:, 0], gpad])
    gx0c = jnp.concatenate([gt[:, 1], gpad])
    gy1c = jnp.concatenate([gt[:, 2], gpad])
    gx1c = jnp.concatenate([gt[:, 3], gpad])
    hw = jnp.concatenate([
        jnp.full((L,), image_shape[0], f32),
        jnp.full((L,), image_shape[1], f32),
    ])

    call = _build_call()
    out = call(ay0, ax0, ay1, ax1, s0, s1, p0, p1, p2, p3, rk,
               gy0s, gx0s, gy1s, gx1s, gabs, gy0c, gx0c, gy1c, gx1c, hw)[0]
    return (out[0], out[1])


# speedup vs baseline: 2.0383x; 2.0383x over previous
import jax
import jax.numpy as jnp
from jax import lax
from jax.experimental import pallas as pl
from jax.experimental.pallas import tpu as pltpu
from jax.experimental.pallas import tpu_sc as plsc

L = 16
NW = 16
N = 20000
NPAD = 20480
NA = NPAD // NW
NV = NA // L
G = 50
GP = 64
JB = 5
HB = 256
NR = 10

POS_T = 0.7
NEG_T = 0.3
TOTAL = 256
MAX_POS = 128
SIG2 = 9.0

_LN_COEFS = (
    -0.0022883228657252968, 0.038030295273843794, -0.2864361250512785,
    1.2917075421662867, -3.8809206183156606, 8.178308102497969,
    -12.396895192830529, 13.666792234339184, -11.06002906824556,
    7.031391849388096, -2.5796606939698807,
)
_LN2 = 0.6931471805599453


def _ln_12(x):
    acc = jnp.full((L,), _LN_COEFS[0], jnp.float32)
    for c in _LN_COEFS[1:]:
        acc = acc * x + jnp.float32(c)
    return acc


def _ln_pos(x):
    bits = plsc.bitcast(x, jnp.int32)
    e = ((bits >> 23) & 0xFF) - 127
    mant = plsc.bitcast((bits & 0x7FFFFF) | 0x3F800000, jnp.float32)
    return e.astype(jnp.float32) * jnp.float32(_LN2) + _ln_12(mant)


def _iota():
    return lax.broadcasted_iota(jnp.int32, (L,), 0)


def _walk(histsum, t):
    def body(c, carry):
        cnt, bacc, sacc = carry
        chunk = histsum[pl.ds(c * L, L)]
        cs = jnp.cumsum(chunk)
        below = cnt + cs - chunk
        is_b = (below < t) & (below + chunk >= t)
        bacc = bacc + jnp.where(is_b, c * L + _iota(), 0)
        sacc = sacc + jnp.where(is_b, below, 0)
        cnt = cnt + jnp.max(cs)
        return cnt, bacc, sacc

    zero = jnp.zeros((L,), jnp.int32)
    _, bacc, sacc = lax.fori_loop(0, HB // L, body,
                                  (jnp.int32(0), zero, zero))
    return jnp.sum(bacc), jnp.sum(sacc)


def _radix_round(r, wid, get_cand, sh, top, flip, pref, t,
                 hist2d, histsum, rdhist, sthist):
    zero = jnp.zeros((L,), jnp.int32)
    ones = jnp.ones((L,), jnp.int32)

    def zbody(i, _):
        hist2d[pl.ds(i * L, L)] = zero
        return 0
    lax.fori_loop(0, HB, zbody, 0)

    io = _iota()

    def scan(v, _):
        mask, key = get_cand(v)
        if not top:
            mask = mask & ((key >> (sh + 8)) == (pref >> (sh + 8)))
        bucket = (key >> sh) & (HB - 1)
        if flip:
            bucket = (HB - 1) - bucket
        slot = io * HB + bucket
        plsc.addupdate_scatter(hist2d, [slot], ones, mask=mask)
        return 0
    lax.fori_loop(0, NV, scan, 0)

    def lred_c(c, _):
        def lred_l(l, acc):
            return acc + hist2d[pl.ds(l * HB + c * L, L)]
        histsum[pl.ds(c * L, L)] = lax.fori_loop(0, L, lred_l, zero)
        return 0
    lax.fori_loop(0, HB // L, lred_c, 0)

    pltpu.sync_copy(histsum, sthist.at[pl.ds(r * NW * HB + wid * HB, HB)])
    plsc.subcore_barrier()
    pltpu.sync_copy(sthist.at[pl.ds(r * NW * HB, NW * HB)], rdhist)

    def gred_c(c, _):
        def gred_i(i, acc):
            return acc + rdhist[pl.ds(i * HB + c * L, L)]
        histsum[pl.ds(c * L, L)] = lax.fori_loop(0, NW, gred_i, zero)
        return 0
    lax.fori_loop(0, HB // L, gred_c, 0)

    bstar, taken = _walk(histsum, t)
    if flip:
        bstar = (HB - 1) - bstar
    return pref | (bstar << sh), t - taken


def _sc_kernel_body(ay0, ax0, ay1, ax1, s0h, s1h, p0h, p1h, p2h, p3h, rkh,
                    gy0s, gx0s, gy1s, gx1s, gabs, gy0c, gx0c, gy1c, gx1c, hwh,
                    out_hbm,
                    cy0, cx0, cy1, cx1, vs0, vs1, vp0, vp1, vp2, vp3, vrk,
                    vgy0, vgx0, vgy1, vgx1, vgab, vg0c, vg1c, vg2c, vg3c, vhw,
                    gtmax, maxiou, bestj, insd, areaa, forced, posm, negm,
                    poskey, hist2d, histsum, rdgt, rdhist, rdsm, wv, outv,
                    st_gtmax, st_cnt, st_hist, st_loss, dsem):
    wid = lax.axis_index("s")
    base = wid * NA
    fzero = jnp.zeros((L,), jnp.float32)
    io = _iota()

    copies = [
        pltpu.async_copy(ay0.at[pl.ds(base, NA)], cy0, dsem),
        pltpu.async_copy(ax0.at[pl.ds(base, NA)], cx0, dsem),
        pltpu.async_copy(ay1.at[pl.ds(base, NA)], cy1, dsem),
        pltpu.async_copy(ax1.at[pl.ds(base, NA)], cx1, dsem),
        pltpu.async_copy(s0h.at[pl.ds(base, NA)], vs0, dsem),
        pltpu.async_copy(s1h.at[pl.ds(base, NA)], vs1, dsem),
        pltpu.async_copy(p0h.at[pl.ds(base, NA)], vp0, dsem),
        pltpu.async_copy(p1h.at[pl.ds(base, NA)], vp1, dsem),
        pltpu.async_copy(p2h.at[pl.ds(base, NA)], vp2, dsem),
        pltpu.async_copy(p3h.at[pl.ds(base, NA)], vp3, dsem),
        pltpu.async_copy(rkh.at[pl.ds(base, NA)], vrk, dsem),
        pltpu.async_copy(gy0s, vgy0, dsem),
        pltpu.async_copy(gx0s, vgx0, dsem),
        pltpu.async_copy(gy1s, vgy1, dsem),
        pltpu.async_copy(gx1s, vgx1, dsem),
        pltpu.async_copy(gabs, vgab, dsem),
        pltpu.async_copy(gy0c, vg0c, dsem),
        pltpu.async_copy(gx0c, vg1c, dsem),
        pltpu.async_copy(gy1c, vg2c, dsem),
        pltpu.async_copy(gx1c, vg3c, dsem),
        pltpu.async_copy(hwh, vhw, dsem),
    ]
    for cp in copies:
        cp.wait()

    hvec = vhw[pl.ds(0, L)]
    wvec = vhw[pl.ds(L, L)]

    def p05(v, _):
        o = v * L
        a0 = cy0[pl.ds(o, L)]
        a1 = cx0[pl.ds(o, L)]
        a2 = cy1[pl.ds(o, L)]
        a3 = cx1[pl.ds(o, L)]
        ins = (a0 >= 0.0) & (a1 >= 0.0) & (a2 <= hvec) & (a3 <= wvec)
        insd[pl.ds(o, L)] = jnp.where(ins, 1.0, 0.0)
        areaa[pl.ds(o, L)] = (a2 - a0) * (a3 - a1)
        maxiou[pl.ds(o, L)] = fzero - 1e30
        bestj[pl.ds(o, L)] = jnp.zeros((L,), jnp.int32)
        forced[pl.ds(o, L)] = fzero
        return 0
    lax.fori_loop(0, NV, p05, 0)

    for b in range(G // JB):
        gd = []
        for jj in range(JB):
            j = b * JB + jj
            gd.append((vgy0[pl.ds(j * L, L)], vgx0[pl.ds(j * L, L)],
                       vgy1[pl.ds(j * L, L)], vgx1[pl.ds(j * L, L)],
                       vgab[pl.ds(j * L, L)]))

        def p1(v, carry):
            gtm = list(carry[0])
            gti = list(carry[1])
            o = v * L
            a0 = cy0[pl.ds(o, L)]
            a1 = cx0[pl.ds(o, L)]
            a2 = cy1[pl.ds(o, L)]
            a3 = cx1[pl.ds(o, L)]
            ins = insd[pl.ds(o, L)] > 0.5
            area_a = areaa[pl.ds(o, L)]
            best = maxiou[pl.ds(o, L)]
            bj = bestj[pl.ds(o, L)]
            oio = o + io
            for jj in range(JB):
                g0, g1, g2, g3, ab = gd[jj]
                ih = jnp.maximum(jnp.minimum(a2, g2) - jnp.maximum(a0, g0),
                                 0.0)
                iw = jnp.maximum(jnp.minimum(a3, g3) - jnp.maximum(a1, g1),
                                 0.0)
                inter = ih * iw
                iou = inter / ((area_a + ab) - inter)
                iou = jnp.where(ins, iou, -1.0)
                upd = iou > best
                best = jnp.where(upd, iou, best)
                bj = jnp.where(upd, b * JB + jj, bj)
                upd2 = iou > gtm[jj]
                gtm[jj] = jnp.where(upd2, iou, gtm[jj])
                gti[jj] = jnp.where(upd2, oio, gti[jj])
            maxiou[pl.ds(o, L)] = best
            bestj[pl.ds(o, L)] = bj
            return tuple(gtm), tuple(gti)

        init = (tuple(fzero - 1e30 for _ in range(JB)),
                tuple(jnp.zeros((L,), jnp.int32) for _ in range(JB)))
        gtm, gti = lax.fori_loop(0, NV, p1, init)
        for jj in range(JB):
            j = b * JB + jj
            gtmax[pl.ds(j * L, L)] = gtm[jj]
            rdgt[pl.ds(j * L, L)] = gti[jj].astype(jnp.float32)

    pltpu.sync_copy(gtmax, st_gtmax.at[pl.ds(wid * G * L, G * L)])
    plsc.subcore_barrier()
    pltpu.sync_copy(st_gtmax, rdgt.at[pl.ds(G * L, NW * G * L)])

    onesf = fzero + 1.0
    for j in range(G):
        def fmax(i, acc):
            return jnp.maximum(
                acc, rdgt[pl.ds(G * L + i * G * L + j * L, L)])
        gm = lax.fori_loop(0, NW, fmax, fzero - 1e30)
        g = jnp.max(gm)
        mine = gtmax[pl.ds(j * L, L)]
        match = (mine >= g) & (mine > 0.0)
        cidx = rdgt[pl.ds(j * L, L)].astype(jnp.int32)
        plsc.store_scatter(forced, [cidx], onesf, mask=match)

    def p2_body(v, carry):
        cp_acc, cn_acc = carry
        o = v * L
        best = maxiou[pl.ds(o, L)]
        ins = insd[pl.ds(o, L)] > 0.5
        fc = forced[pl.ds(o, L)] > 0.5
        pm = ins & ((best >= POS_T) | fc)
        nm = ins & (best < NEG_T) & (best >= 0.0)
        posm[pl.ds(o, L)] = jnp.where(pm, 1.0, 0.0)
        negm[pl.ds(o, L)] = jnp.where(nm, 1.0, 0.0)
        key = plsc.bitcast(best, jnp.int32)
        poskey[pl.ds(o, L)] = jnp.where(pm, key, 0)
        return (cp_acc + jnp.where(pm, 1.0, 0.0),
                cn_acc + jnp.where(nm, 1.0, 0.0))

    cp_acc, cn_acc = lax.fori_loop(0, NV, p2_body, (fzero, fzero))
    wv[pl.ds(0, L)] = cp_acc
    wv[pl.ds(L, L)] = cn_acc
    pltpu.sync_copy(wv, st_cnt.at[pl.ds(wid * 2 * L, 2 * L)])
    plsc.subcore_barrier()
    pltpu.sync_copy(st_cnt, rdsm)

    def cmerge(i, carry):
        a, b2 = carry
        return (a + rdsm[pl.ds(i * 2 * L, L)],
                b2 + rdsm[pl.ds(i * 2 * L + L, L)])
    cpv, cnv = lax.fori_loop(0, NW, cmerge, (fzero, fzero))
    cnt_pos = jnp.sum(cpv)
    cnt_neg = jnp.sum(cnv)
    n_pos = jnp.minimum(cnt_pos, float(MAX_POS))
    need_f = float(TOTAL) - n_pos
    cp_i = cnt_pos.astype(jnp.int32)
    cn_i = cnt_neg.astype(jnp.int32)
    need_i = jnp.int32(TOTAL) - jnp.minimum(cp_i, MAX_POS)
    kfill_i = jnp.maximum(need_i - cn_i, 0)
    pos_over = cnt_pos > float(MAX_POS)
    neg_over = cnt_neg > need_f

    def get_pos(v):
        o = v * L
        return posm[pl.ds(o, L)] > 0.5, poskey[pl.ds(o, L)]

    def get_neg(v):
        o = v * L
        return negm[pl.ds(o, L)] > 0.5, vrk[pl.ds(o, L)]

    def rrnd(r, get, sh, top, flip, pref, t):
        return _radix_round(r, wid, get, sh, top, flip, pref, t,
                            hist2d, histsum, rdhist, st_hist)

    def pos_rounds(_):
        pref, t = rrnd(0, get_pos, 24, True, True, jnp.int32(0),
                       jnp.int32(MAX_POS))
        pref, t = rrnd(1, get_pos, 16, False, True, pref, t)
        pref, t = rrnd(2, get_pos, 8, False, True, pref, t)
        pref, t = rrnd(3, get_pos, 0, False, True, pref, t)
        kv_, trem = pref, t

        def get_tie(v):
            o = v * L
            m = (posm[pl.ds(o, L)] > 0.5) & (poskey[pl.ds(o, L)] == kv_)
            return m, base + o + io

        pref2, t2 = rrnd(4, get_tie, 8, True, False, jnp.int32(0), trem)
        pref2, _ = rrnd(5, get_tie, 0, False, False, pref2, t2)
        return kv_, pref2

    kv, ki = lax.cond(pos_over, pos_rounds,
                      lambda _: (jnp.int32(0), jnp.int32(NPAD)),
                      0)

    def neg_rounds(_):
        pref3, t3 = rrnd(6, get_neg, 8, True, False, jnp.int32(0), need_i)
        pref3, _ = rrnd(7, get_neg, 0, False, False, pref3, t3)
        return pref3

    kr = lax.cond(neg_over, neg_rounds, lambda _: jnp.int32(NPAD + 1), 0)

    def get_fill(v):
        o = v * L
        idx = wid * NA + o + io
        m = (negm[pl.ds(o, L)] <= 0.5) & (idx < N)
        return m, idx

    def fill_rounds(_):
        pref4, t4 = rrnd(8, get_fill, 8, True, False, jnp.int32(0), kfill_i)
        pref4, _ = rrnd(9, get_fill, 0, False, False, pref4, t4)
        return pref4

    kf = lax.cond(kfill_i > 0, fill_rounds, lambda _: jnp.int32(-1), 0)

    def p4_body(v, carry):
        acc_cls, acc_reg = carry
        o = v * L
        idx = base + o + io
        pm = posm[pl.ds(o, L)] > 0.5
        nm = negm[pl.ds(o, L)] > 0.5
        key = poskey[pl.ds(o, L)]
        rk = vrk[pl.ds(o, L)]
        sel_pos = pm & ((key > kv) | ((key == kv) & (idx <= ki)))
        sel_neg = nm & (rk <= kr)
        sel_fill = (~nm) & (idx < N) & (idx <= kf)
        neg_w = sel_neg | sel_fill

        sv0 = vs0[pl.ds(o, L)]
        sv1 = vs1[pl.ds(o, L)]
        m = jnp.maximum(sv0, sv1)
        esum = jnp.exp(sv0 - m) + jnp.exp(sv1 - m)
        lse = m + _ln_12(esum)
        lp0 = sv0 - lse
        lp1 = sv1 - lse
        acc_cls = (acc_cls - jnp.where(sel_pos, lp1, 0.0)
                   - jnp.where(neg_w, lp0, 0.0))

        a0 = cy0[pl.ds(o, L)]
        a1 = cx0[pl.ds(o, L)]
        a2 = cy1[pl.ds(o, L)]
        a3 = cx1[pl.ds(o, L)]
        bj = bestj[pl.ds(o, L)]
        g0 = plsc.load_gather(vg0c, [bj])
        g1 = plsc.load_gather(vg1c, [bj])
        g2 = plsc.load_gather(vg2c, [bj])
        g3 = plsc.load_gather(vg3c, [bj])
        a_h = a2 - a0
        a_w = a3 - a1
        a_cy = a0 + 0.5 * a_h
        a_cx = a1 + 0.5 * a_w
        g_h = g2 - g0
        g_w = g3 - g1
        g_cy = g0 + 0.5 * g_h
        g_cx = g1 + 0.5 * g_w
        eps = jnp.float32(1e-8)
        ty = (g_cy - a_cy) / (a_h + eps)
        tx = (g_cx - a_cx) / (a_w + eps)
        th = _ln_pos(jnp.maximum(g_h, eps)) - _ln_pos(jnp.maximum(a_h, eps))
        tw = _ln_pos(jnp.maximum(g_w, eps)) - _ln_pos(jnp.maximum(a_w, eps))
        ssum = fzero
        for pref_ref, tgt in ((vp0, tx), (vp1, ty), (vp2, tw), (vp3, th)):
            d = pref_ref[pl.ds(o, L)] - tgt
            ad = jnp.abs(d)
            sl = jnp.where(ad < 1.0 / SIG2, 0.5 * SIG2 * d * d,
                           ad - 0.5 / SIG2)
            ssum = ssum + sl
        acc_reg = acc_reg + jnp.where(sel_pos, ssum, 0.0)
        return acc_cls, acc_reg

    acc_cls, acc_reg = lax.fori_loop(0, NV, p4_body, (fzero, fzero))
    wv[pl.ds(0, L)] = acc_cls
    wv[pl.ds(L, L)] = acc_reg
    pltpu.sync_copy(wv, st_loss.at[pl.ds(wid * 2 * L, 2 * L)])
    plsc.subcore_barrier()
    pltpu.sync_copy(st_loss, rdsm)

    def lmerge(i, carry):
        a, b2 = carry
        return (a + rdsm[pl.ds(i * 2 * L, L)],
                b2 + rdsm[pl.ds(i * 2 * L + L, L)])
    av, bv = lax.fori_loop(0, NW, lmerge, (fzero, fzero))
    cls_sum = jnp.sum(av)
    reg_sum = jnp.sum(bv)
    numer = (jnp.where(io == 0, cls_sum, 0.0)
             + jnp.where(io == 1, reg_sum, 0.0))
    denom = jnp.where(io == 1, jnp.maximum(n_pos, 1.0),
                      jnp.float32(TOTAL))
    outv[pl.ds(0, L)] = numer / denom

    @pl.when(wid == 0)
    def _():
        pltpu.sync_copy(outv, out_hbm)


def _build_call():
    mesh = plsc.VectorSubcoreMesh(core_axis_name="c", subcore_axis_name="s",
                                  num_cores=1, num_subcores=NW)
    f32, i32 = jnp.float32, jnp.int32
    return pl.kernel(
        _sc_kernel_body,
        out_type=[
            jax.ShapeDtypeStruct((L,), f32),
        ],
        mesh=mesh,
        compiler_params=pltpu.CompilerParams(needs_layout_passes=False),
        scratch_types=[
            pltpu.VMEM((NA,), f32),
            pltpu.VMEM((NA,), f32),
            pltpu.VMEM((NA,), f32),
            pltpu.VMEM((NA,), f32),
            pltpu.VMEM((NA,), f32),
            pltpu.VMEM((NA,), f32),
            pltpu.VMEM((NA,), f32),
            pltpu.VMEM((NA,), f32),
            pltpu.VMEM((NA,), f32),
            pltpu.VMEM((NA,), f32),
            pltpu.VMEM((NA,), i32),
            pltpu.VMEM((G * L,), f32),
            pltpu.VMEM((G * L,), f32),
            pltpu.VMEM((G * L,), f32),
            pltpu.VMEM((G * L,), f32),
            pltpu.VMEM((G * L,), f32),
            pltpu.VMEM((GP,), f32),
            pltpu.VMEM((GP,), f32),
            pltpu.VMEM((GP,), f32),
            pltpu.VMEM((GP,), f32),
            pltpu.VMEM((2 * L,), f32),
            pltpu.VMEM((G * L,), f32),
            pltpu.VMEM((NA,), f32),
            pltpu.VMEM((NA,), i32),
            pltpu.VMEM((NA,), f32),
            pltpu.VMEM((NA,), f32),
            pltpu.VMEM((NA,), f32),
            pltpu.VMEM((NA,), f32),
            pltpu.VMEM((NA,), f32),
            pltpu.VMEM((NA,), i32),
            pltpu.VMEM((L * HB,), i32),
            pltpu.VMEM((HB,), i32),
            pltpu.VMEM(((NW + 1) * G * L,), f32),
            pltpu.VMEM((NW * HB,), i32),
            pltpu.VMEM((NW * 2 * L,), f32),
            pltpu.VMEM((2 * L,), f32),
            pltpu.VMEM((L,), f32),
            pltpu.VMEM_SHARED((NW * G * L,), f32),
            pltpu.VMEM_SHARED((NW * 2 * L,), f32),
            pltpu.VMEM_SHARED((NR * NW * HB,), i32),
            pltpu.VMEM_SHARED((NW * 2 * L,), f32),
            pltpu.SemaphoreType.DMA,
        ],
    )


def _pad1(x, value):
    return jnp.concatenate(
        [x, jnp.full((NPAD - N,), value, x.dtype)])


def kernel(image_shape, anchors, rpn_score, rpn_bboxes_txtytwth, gt_bboxes):
    f32 = jnp.float32
    rngv = jax.random.uniform(jax.random.key(123), (N,))
    order = jnp.argsort(-rngv, stable=True)
    rank = jnp.argsort(order, stable=True).astype(jnp.int32)

    ay0 = _pad1(anchors[:, 0].astype(f32), -1.0)
    ax0 = _pad1(anchors[:, 1].astype(f32), -1.0)
    ay1 = _pad1(anchors[:, 2].astype(f32), -1.0)
    ax1 = _pad1(anchors[:, 3].astype(f32), -1.0)
    s0 = _pad1(rpn_score[:, 0].astype(f32), 0.0)
    s1 = _pad1(rpn_score[:, 1].astype(f32), 0.0)
    p0 = _pad1(rpn_bboxes_txtytwth[:, 0].astype(f32), 0.0)
    p1 = _pad1(rpn_bboxes_txtytwth[:, 1].astype(f32), 0.0)
    p2 = _pad1(rpn_bboxes_txtytwth[:, 2].astype(f32), 0.0)
    p3 = _pad1(rpn_bboxes_txtytwth[:, 3].astype(f32), 0.0)
    rk = _pad1(rank, 32000)

    gt = gt_bboxes.astype(f32)
    gy0s = jnp.broadcast_to(gt[:, 0:1], (G, L)).reshape(-1)
    gx0s = jnp.broadcast_to(gt[:, 1:2], (G, L)).reshape(-1)
    gy1s = jnp.broadcast_to(gt[:, 2:3], (G, L)).reshape(-1)
    gx1s = jnp.broadcast_to(gt[:, 3:4], (G, L)).reshape(-1)
    ab = ((gt[:, 2] - gt[:, 0]) * (gt[:, 3] - gt[:, 1]) + 1e-9)
    gabs = jnp.broadcast_to(ab[:, None], (G, L)).reshape(-1)
    gpad = jnp.zeros((GP - G,), f32)
    gy0c = jnp.concatenate([gt[:, 0], gpad])
    gx0c = jnp.concatenate([gt[:, 1], gpad])
    gy1c = jnp.concatenate([gt[:, 2], gpad])
    gx1c = jnp.concatenate([gt[:, 3], gpad])
    hw = jnp.concatenate([
        jnp.full((L,), image_shape[0], f32),
        jnp.full((L,), image_shape[1], f32),
    ])

    call = _build_call()
    out = call(ay0, ax0, ay1, ax1, s0, s1, p0, p1, p2, p3, rk,
               gy0s, gx0s, gy1s, gx1s, gabs, gy0c, gx0c, gy1c, gx1c, hw)[0]
    return (out[0], out[1])


# speedup vs baseline: 2.6785x; 1.3141x over previous
import jax
import jax.numpy as jnp
from jax import lax
from jax.experimental import pallas as pl
from jax.experimental.pallas import tpu as pltpu
from jax.experimental.pallas import tpu_sc as plsc

L = 16
NW = 16
N = 20000
NPAD = 20480
NA = NPAD // NW
NV = NA // L
G = 50
GP = 64
JB = 5
HB = 256
NR = 10

POS_T = 0.7
NEG_T = 0.3
TOTAL = 256
MAX_POS = 128
SIG2 = 9.0

_LN_COEFS = (
    -0.0022883228657252968, 0.038030295273843794, -0.2864361250512785,
    1.2917075421662867, -3.8809206183156606, 8.178308102497969,
    -12.396895192830529, 13.666792234339184, -11.06002906824556,
    7.031391849388096, -2.5796606939698807,
)
_LN2 = 0.6931471805599453


def _ln_12(x):
    acc = jnp.full((L,), _LN_COEFS[0], jnp.float32)
    for c in _LN_COEFS[1:]:
        acc = acc * x + jnp.float32(c)
    return acc


def _ln_pos(x):
    bits = plsc.bitcast(x, jnp.int32)
    e = ((bits >> 23) & 0xFF) - 127
    mant = plsc.bitcast((bits & 0x7FFFFF) | 0x3F800000, jnp.float32)
    return e.astype(jnp.float32) * jnp.float32(_LN2) + _ln_12(mant)


def _iota():
    return lax.broadcasted_iota(jnp.int32, (L,), 0)


def _walk(histsum, t):
    def body(c, carry):
        cnt, bacc, sacc = carry
        chunk = histsum[pl.ds(c * L, L)]
        cs = jnp.cumsum(chunk)
        below = cnt + cs - chunk
        is_b = (below < t) & (below + chunk >= t)
        bacc = bacc + jnp.where(is_b, c * L + _iota(), 0)
        sacc = sacc + jnp.where(is_b, below, 0)
        cnt = cnt + jnp.max(cs)
        return cnt, bacc, sacc

    zero = jnp.zeros((L,), jnp.int32)
    _, bacc, sacc = lax.fori_loop(0, HB // L, body,
                                  (jnp.int32(0), zero, zero))
    return jnp.sum(bacc), jnp.sum(sacc)


def _radix_round(r, wid, get_cand, sh, top, flip, pref, t,
                 hist2d, histsum, rdhist, sthist):
    zero = jnp.zeros((L,), jnp.int32)
    ones = jnp.ones((L,), jnp.int32)

    def zbody(i, _):
        hist2d[pl.ds(i * L, L)] = zero
        return 0
    lax.fori_loop(0, HB, zbody, 0)

    io = _iota()

    def scan(v, _):
        mask, key = get_cand(v)
        if not top:
            mask = mask & ((key >> (sh + 8)) == (pref >> (sh + 8)))
        bucket = (key >> sh) & (HB - 1)
        if flip:
            bucket = (HB - 1) - bucket
        slot = io * HB + bucket
        plsc.addupdate_scatter(hist2d, [slot], ones, mask=mask)
        return 0
    lax.fori_loop(0, NV, scan, 0)

    def lred_c(c, _):
        def lred_l(l, acc):
            return acc + hist2d[pl.ds(l * HB + c * L, L)]
        histsum[pl.ds(c * L, L)] = lax.fori_loop(0, L, lred_l, zero)
        return 0
    lax.fori_loop(0, HB // L, lred_c, 0)

    pltpu.sync_copy(histsum, sthist.at[pl.ds(r * NW * HB + wid * HB, HB)])
    plsc.subcore_barrier()
    pltpu.sync_copy(sthist.at[pl.ds(r * NW * HB, NW * HB)], rdhist)

    def gred_c(c, _):
        def gred_i(i, acc):
            return acc + rdhist[pl.ds(i * HB + c * L, L)]
        histsum[pl.ds(c * L, L)] = lax.fori_loop(0, NW, gred_i, zero)
        return 0
    lax.fori_loop(0, HB // L, gred_c, 0)

    bstar, taken = _walk(histsum, t)
    if flip:
        bstar = (HB - 1) - bstar
    return pref | (bstar << sh), t - taken


_GO_Y0 = 0
_GO_X0 = G * L
_GO_Y1 = 2 * G * L
_GO_X1 = 3 * G * L
_GO_AB = 4 * G * L
_GO_C0 = 5 * G * L
_GO_C1 = 5 * G * L + GP
_GO_C2 = 5 * G * L + 2 * GP
_GO_C3 = 5 * G * L + 3 * GP
_GO_HW = 5 * G * L + 4 * GP
GTB = _GO_HW + 2 * L


def _sc_kernel_body(xh, gtbh,
                    out_hbm,
                    cy0, cx0, cy1, cx1, vs0, vs1, vp0, vp1, vp2, vp3, vrkf,
                    gtb,
                    gtmax, maxiou, bestj, insd, areaa, forced, posm, negm,
                    poskey, hist2d, histsum, rdgt, rdhist, rdsm, wv, outv,
                    st_gtmax, st_cnt, st_hist, st_loss, dsem):
    wid = lax.axis_index("s")
    base = wid * NA
    fzero = jnp.zeros((L,), jnp.float32)
    io = _iota()

    dsts = [cy0, cx0, cy1, cx1, vs0, vs1, vp0, vp1, vp2, vp3, vrkf]
    copies = [
        pltpu.async_copy(xh.at[pl.ds(r * NPAD + base, NA)], dst, dsem)
        for r, dst in enumerate(dsts)
    ]
    copies.append(pltpu.async_copy(gtbh, gtb, dsem))
    for cp in copies:
        cp.wait()

    hvec = gtb[pl.ds(_GO_HW, L)]
    wvec = gtb[pl.ds(_GO_HW + L, L)]

    def p05(v, _):
        o = v * L
        a0 = cy0[pl.ds(o, L)]
        a1 = cx0[pl.ds(o, L)]
        a2 = cy1[pl.ds(o, L)]
        a3 = cx1[pl.ds(o, L)]
        ins = (a0 >= 0.0) & (a1 >= 0.0) & (a2 <= hvec) & (a3 <= wvec)
        insd[pl.ds(o, L)] = jnp.where(ins, 1.0, 0.0)
        areaa[pl.ds(o, L)] = (a2 - a0) * (a3 - a1)
        maxiou[pl.ds(o, L)] = fzero - 1e30
        bestj[pl.ds(o, L)] = jnp.zeros((L,), jnp.int32)
        forced[pl.ds(o, L)] = fzero
        return 0
    lax.fori_loop(0, NV, p05, 0)

    for b in range(G // JB):
        gd = []
        for jj in range(JB):
            j = b * JB + jj
            gd.append((gtb[pl.ds(_GO_Y0 + j * L, L)],
                       gtb[pl.ds(_GO_X0 + j * L, L)],
                       gtb[pl.ds(_GO_Y1 + j * L, L)],
                       gtb[pl.ds(_GO_X1 + j * L, L)],
                       gtb[pl.ds(_GO_AB + j * L, L)]))

        def p1(v, carry):
            gtm = list(carry[0])
            gti = list(carry[1])
            o = v * L
            a0 = cy0[pl.ds(o, L)]
            a1 = cx0[pl.ds(o, L)]
            a2 = cy1[pl.ds(o, L)]
            a3 = cx1[pl.ds(o, L)]
            ins = insd[pl.ds(o, L)] > 0.5
            area_a = areaa[pl.ds(o, L)]
            best = maxiou[pl.ds(o, L)]
            bj = bestj[pl.ds(o, L)]
            oio = o + io
            for jj in range(JB):
                g0, g1, g2, g3, ab = gd[jj]
                ih = jnp.maximum(jnp.minimum(a2, g2) - jnp.maximum(a0, g0),
                                 0.0)
                iw = jnp.maximum(jnp.minimum(a3, g3) - jnp.maximum(a1, g1),
                                 0.0)
                inter = ih * iw
                iou = inter / ((area_a + ab) - inter)
                iou = jnp.where(ins, iou, -1.0)
                upd = iou > best
                best = jnp.where(upd, iou, best)
                bj = jnp.where(upd, b * JB + jj, bj)
                upd2 = iou > gtm[jj]
                gtm[jj] = jnp.where(upd2, iou, gtm[jj])
                gti[jj] = jnp.where(upd2, oio, gti[jj])
            maxiou[pl.ds(o, L)] = best
            bestj[pl.ds(o, L)] = bj
            return tuple(gtm), tuple(gti)

        init = (tuple(fzero - 1e30 for _ in range(JB)),
                tuple(jnp.zeros((L,), jnp.int32) for _ in range(JB)))
        gtm, gti = lax.fori_loop(0, NV, p1, init)
        for jj in range(JB):
            j = b * JB + jj
            gtmax[pl.ds(j * L, L)] = gtm[jj]
            rdgt[pl.ds(j * L, L)] = gti[jj].astype(jnp.float32)

    pltpu.sync_copy(gtmax, st_gtmax.at[pl.ds(wid * G * L, G * L)])
    plsc.subcore_barrier()
    pltpu.sync_copy(st_gtmax, rdgt.at[pl.ds(G * L, NW * G * L)])

    onesf = fzero + 1.0
    for j in range(G):
        def fmax(i, acc):
            return jnp.maximum(
                acc, rdgt[pl.ds(G * L + i * G * L + j * L, L)])
        gm = lax.fori_loop(0, NW, fmax, fzero - 1e30)
        g = jnp.max(gm)
        mine = gtmax[pl.ds(j * L, L)]
        match = (mine >= g) & (mine > 0.0)
        cidx = rdgt[pl.ds(j * L, L)].astype(jnp.int32)
        plsc.store_scatter(forced, [cidx], onesf, mask=match)

    def p2_body(v, carry):
        cp_acc, cn_acc = carry
        o = v * L
        best = maxiou[pl.ds(o, L)]
        ins = insd[pl.ds(o, L)] > 0.5
        fc = forced[pl.ds(o, L)] > 0.5
        pm = ins & ((best >= POS_T) | fc)
        nm = ins & (best < NEG_T) & (best >= 0.0)
        posm[pl.ds(o, L)] = jnp.where(pm, 1.0, 0.0)
        negm[pl.ds(o, L)] = jnp.where(nm, 1.0, 0.0)
        key = plsc.bitcast(best, jnp.int32)
        poskey[pl.ds(o, L)] = jnp.where(pm, key, 0)
        return (cp_acc + jnp.where(pm, 1.0, 0.0),
                cn_acc + jnp.where(nm, 1.0, 0.0))

    cp_acc, cn_acc = lax.fori_loop(0, NV, p2_body, (fzero, fzero))
    wv[pl.ds(0, L)] = cp_acc
    wv[pl.ds(L, L)] = cn_acc
    pltpu.sync_copy(wv, st_cnt.at[pl.ds(wid * 2 * L, 2 * L)])
    plsc.subcore_barrier()
    pltpu.sync_copy(st_cnt, rdsm)

    def cmerge(i, carry):
        a, b2 = carry
        return (a + rdsm[pl.ds(i * 2 * L, L)],
                b2 + rdsm[pl.ds(i * 2 * L + L, L)])
    cpv, cnv = lax.fori_loop(0, NW, cmerge, (fzero, fzero))
    cnt_pos = jnp.sum(cpv)
    cnt_neg = jnp.sum(cnv)
    n_pos = jnp.minimum(cnt_pos, float(MAX_POS))
    need_f = float(TOTAL) - n_pos
    cp_i = cnt_pos.astype(jnp.int32)
    cn_i = cnt_neg.astype(jnp.int32)
    need_i = jnp.int32(TOTAL) - jnp.minimum(cp_i, MAX_POS)
    kfill_i = jnp.maximum(need_i - cn_i, 0)
    pos_over = cnt_pos > float(MAX_POS)
    neg_over = cnt_neg > need_f

    def get_pos(v):
        o = v * L
        return posm[pl.ds(o, L)] > 0.5, poskey[pl.ds(o, L)]

    def get_neg(v):
        o = v * L
        return (negm[pl.ds(o, L)] > 0.5,
                plsc.bitcast(vrkf[pl.ds(o, L)], jnp.int32))

    def rrnd(r, get, sh, top, flip, pref, t):
        return _radix_round(r, wid, get, sh, top, flip, pref, t,
                            hist2d, histsum, rdhist, st_hist)

    def pos_rounds(_):
        pref, t = rrnd(0, get_pos, 24, True, True, jnp.int32(0),
                       jnp.int32(MAX_POS))
        pref, t = rrnd(1, get_pos, 16, False, True, pref, t)
        pref, t = rrnd(2, get_pos, 8, False, True, pref, t)
        pref, t = rrnd(3, get_pos, 0, False, True, pref, t)
        kv_, trem = pref, t

        def get_tie(v):
            o = v * L
            m = (posm[pl.ds(o, L)] > 0.5) & (poskey[pl.ds(o, L)] == kv_)
            return m, base + o + io

        pref2, t2 = rrnd(4, get_tie, 8, True, False, jnp.int32(0), trem)
        pref2, _ = rrnd(5, get_tie, 0, False, False, pref2, t2)
        return kv_, pref2

    kv, ki = lax.cond(pos_over, pos_rounds,
                      lambda _: (jnp.int32(0), jnp.int32(NPAD)),
                      0)

    def neg_rounds(_):
        pref3, t3 = rrnd(6, get_neg, 8, True, False, jnp.int32(0), need_i)
        pref3, _ = rrnd(7, get_neg, 0, False, False, pref3, t3)
        return pref3

    kr = lax.cond(neg_over, neg_rounds, lambda _: jnp.int32(NPAD + 1), 0)

    def get_fill(v):
        o = v * L
        idx = wid * NA + o + io
        m = (negm[pl.ds(o, L)] <= 0.5) & (idx < N)
        return m, idx

    def fill_rounds(_):
        pref4, t4 = rrnd(8, get_fill, 8, True, False, jnp.int32(0), kfill_i)
        pref4, _ = rrnd(9, get_fill, 0, False, False, pref4, t4)
        return pref4

    kf = lax.cond(kfill_i > 0, fill_rounds, lambda _: jnp.int32(-1), 0)

    def p4_body(v, carry):
        acc_cls, acc_reg = carry
        o = v * L
        idx = base + o + io
        pm = posm[pl.ds(o, L)] > 0.5
        nm = negm[pl.ds(o, L)] > 0.5
        key = poskey[pl.ds(o, L)]
        rk = plsc.bitcast(vrkf[pl.ds(o, L)], jnp.int32)
        sel_pos = pm & ((key > kv) | ((key == kv) & (idx <= ki)))
        sel_neg = nm & (rk <= kr)
        sel_fill = (~nm) & (idx < N) & (idx <= kf)
        neg_w = sel_neg | sel_fill

        sv0 = vs0[pl.ds(o, L)]
        sv1 = vs1[pl.ds(o, L)]
        m = jnp.maximum(sv0, sv1)
        esum = jnp.exp(sv0 - m) + jnp.exp(sv1 - m)
        lse = m + _ln_12(esum)
        lp0 = sv0 - lse
        lp1 = sv1 - lse
        acc_cls = (acc_cls - jnp.where(sel_pos, lp1, 0.0)
                   - jnp.where(neg_w, lp0, 0.0))

        a0 = cy0[pl.ds(o, L)]
        a1 = cx0[pl.ds(o, L)]
        a2 = cy1[pl.ds(o, L)]
        a3 = cx1[pl.ds(o, L)]
        bj = bestj[pl.ds(o, L)]
        g0 = plsc.load_gather(gtb, [_GO_C0 + bj])
        g1 = plsc.load_gather(gtb, [_GO_C1 + bj])
        g2 = plsc.load_gather(gtb, [_GO_C2 + bj])
        g3 = plsc.load_gather(gtb, [_GO_C3 + bj])
        a_h = a2 - a0
        a_w = a3 - a1
        a_cy = a0 + 0.5 * a_h
        a_cx = a1 + 0.5 * a_w
        g_h = g2 - g0
        g_w = g3 - g1
        g_cy = g0 + 0.5 * g_h
        g_cx = g1 + 0.5 * g_w
        eps = jnp.float32(1e-8)
        ty = (g_cy - a_cy) / (a_h + eps)
        tx = (g_cx - a_cx) / (a_w + eps)
        th = _ln_pos(jnp.maximum(g_h, eps)) - _ln_pos(jnp.maximum(a_h, eps))
        tw = _ln_pos(jnp.maximum(g_w, eps)) - _ln_pos(jnp.maximum(a_w, eps))
        ssum = fzero
        for pref_ref, tgt in ((vp0, tx), (vp1, ty), (vp2, tw), (vp3, th)):
            d = pref_ref[pl.ds(o, L)] - tgt
            ad = jnp.abs(d)
            sl = jnp.where(ad < 1.0 / SIG2, 0.5 * SIG2 * d * d,
                           ad - 0.5 / SIG2)
            ssum = ssum + sl
        acc_reg = acc_reg + jnp.where(sel_pos, ssum, 0.0)
        return acc_cls, acc_reg

    acc_cls, acc_reg = lax.fori_loop(0, NV, p4_body, (fzero, fzero))
    wv[pl.ds(0, L)] = acc_cls
    wv[pl.ds(L, L)] = acc_reg
    pltpu.sync_copy(wv, st_loss.at[pl.ds(wid * 2 * L, 2 * L)])
    plsc.subcore_barrier()
    pltpu.sync_copy(st_loss, rdsm)

    def lmerge(i, carry):
        a, b2 = carry
        return (a + rdsm[pl.ds(i * 2 * L, L)],
                b2 + rdsm[pl.ds(i * 2 * L + L, L)])
    av, bv = lax.fori_loop(0, NW, lmerge, (fzero, fzero))
    cls_sum = jnp.sum(av)
    reg_sum = jnp.sum(bv)
    numer = (jnp.where(io == 0, cls_sum, 0.0)
             + jnp.where(io == 1, reg_sum, 0.0))
    denom = jnp.where(io == 1, jnp.maximum(n_pos, 1.0),
                      jnp.float32(TOTAL))
    outv[pl.ds(0, L)] = numer / denom

    @pl.when(wid == 0)
    def _():
        pltpu.sync_copy(outv, out_hbm)


def _build_call():
    mesh = plsc.VectorSubcoreMesh(core_axis_name="c", subcore_axis_name="s",
                                  num_cores=1, num_subcores=NW)
    f32, i32 = jnp.float32, jnp.int32
    return pl.kernel(
        _sc_kernel_body,
        out_type=[
            jax.ShapeDtypeStruct((L,), f32),
        ],
        mesh=mesh,
        compiler_params=pltpu.CompilerParams(needs_layout_passes=False),
        scratch_types=[
            pltpu.VMEM((NA,), f32),
            pltpu.VMEM((NA,), f32),
            pltpu.VMEM((NA,), f32),
            pltpu.VMEM((NA,), f32),
            pltpu.VMEM((NA,), f32),
            pltpu.VMEM((NA,), f32),
            pltpu.VMEM((NA,), f32),
            pltpu.VMEM((NA,), f32),
            pltpu.VMEM((NA,), f32),
            pltpu.VMEM((NA,), f32),
            pltpu.VMEM((NA,), f32),
            pltpu.VMEM((GTB,), f32),
            pltpu.VMEM((G * L,), f32),
            pltpu.VMEM((NA,), f32),
            pltpu.VMEM((NA,), i32),
            pltpu.VMEM((NA,), f32),
            pltpu.VMEM((NA,), f32),
            pltpu.VMEM((NA,), f32),
            pltpu.VMEM((NA,), f32),
            pltpu.VMEM((NA,), f32),
            pltpu.VMEM((NA,), i32),
            pltpu.VMEM((L * HB,), i32),
            pltpu.VMEM((HB,), i32),
            pltpu.VMEM(((NW + 1) * G * L,), f32),
            pltpu.VMEM((NW * HB,), i32),
            pltpu.VMEM((NW * 2 * L,), f32),
            pltpu.VMEM((2 * L,), f32),
            pltpu.VMEM((L,), f32),
            pltpu.VMEM_SHARED((NW * G * L,), f32),
            pltpu.VMEM_SHARED((NW * 2 * L,), f32),
            pltpu.VMEM_SHARED((NR * NW * HB,), i32),
            pltpu.VMEM_SHARED((NW * 2 * L,), f32),
            pltpu.SemaphoreType.DMA,
        ],
    )


def _pad1(x, value):
    return jnp.concatenate(
        [x, jnp.full((NPAD - N,), value, x.dtype)])


def kernel(image_shape, anchors, rpn_score, rpn_bboxes_txtytwth, gt_bboxes):
    f32 = jnp.float32
    with jax.ensure_compile_time_eval():
        rngv = jax.random.uniform(jax.random.key(123), (N,))
        order = jnp.argsort(-rngv, stable=True)
        rank = jnp.argsort(order, stable=True).astype(jnp.int32)
        rkf = lax.bitcast_convert_type(_pad1(rank, 32000), f32)

    ay0 = _pad1(anchors[:, 0].astype(f32), -1.0)
    ax0 = _pad1(anchors[:, 1].astype(f32), -1.0)
    ay1 = _pad1(anchors[:, 2].astype(f32), -1.0)
    ax1 = _pad1(anchors[:, 3].astype(f32), -1.0)
    s0 = _pad1(rpn_score[:, 0].astype(f32), 0.0)
    s1 = _pad1(rpn_score[:, 1].astype(f32), 0.0)
    p0 = _pad1(rpn_bboxes_txtytwth[:, 0].astype(f32), 0.0)
    p1 = _pad1(rpn_bboxes_txtytwth[:, 1].astype(f32), 0.0)
    p2 = _pad1(rpn_bboxes_txtytwth[:, 2].astype(f32), 0.0)
    p3 = _pad1(rpn_bboxes_txtytwth[:, 3].astype(f32), 0.0)
    x = jnp.concatenate([ay0, ax0, ay1, ax1, s0, s1, p0, p1, p2, p3, rkf])

    gt = gt_bboxes.astype(f32)
    ab = ((gt[:, 2] - gt[:, 0]) * (gt[:, 3] - gt[:, 1]) + 1e-9)
    gpad = jnp.zeros((GP - G,), f32)
    gtbuf = jnp.concatenate([
        jnp.broadcast_to(gt[:, 0:1], (G, L)).reshape(-1),
        jnp.broadcast_to(gt[:, 1:2], (G, L)).reshape(-1),
        jnp.broadcast_to(gt[:, 2:3], (G, L)).reshape(-1),
        jnp.broadcast_to(gt[:, 3:4], (G, L)).reshape(-1),
        jnp.broadcast_to(ab[:, None], (G, L)).reshape(-1),
        gt[:, 0], gpad, gt[:, 1], gpad, gt[:, 2], gpad, gt[:, 3], gpad,
        jnp.full((L,), image_shape[0], f32),
        jnp.full((L,), image_shape[1], f32),
    ])

    call = _build_call()
    out = call(x, gtbuf)[0]
    return (out[0], out[1])
